# baseline (speedup 1.0000x reference)
import functools

import jax
import jax.numpy as jnp
from jax import lax
from jax.experimental import pallas as pl
from jax.experimental.pallas import tpu as pltpu
from jax.experimental.pallas import tpu_sc as plsc

DIM = 64
PROMPT_LEN = 20
BATCH = 4096
D = PROMPT_LEN * DIM

_NC = 2
_NS = 16
_RROWS = 256
_C = _RROWS // _NS
_HALF = BATCH // _NC
_NR = _HALF // _RROWS
_SLAB = 128
_NSLAB = D // _SLAB
_Q = 128
_NQ = _RROWS // _Q
_TP = _Q + 1


def _make_gather():
    mesh = plsc.VectorSubcoreMesh(
        core_axis_name="c", subcore_axis_name="s", num_cores=_NC)

    @functools.partial(
        pl.kernel,
        mesh=mesh,
        out_type=jax.ShapeDtypeStruct((D, BATCH), jnp.float32),
        compiler_params=pltpu.CompilerParams(
            needs_layout_passes=False, internal_scratch_in_bytes=131072),
        scratch_types=[
            pltpu.VMEM((_NR, _C), jnp.int32),
            pltpu.VMEM((_C, D), jnp.float32),
            pltpu.VMEM((_Q, _SLAB), jnp.float32),
            pltpu.VMEM((_SLAB, _TP), jnp.float32),
            pltpu.VMEM_SHARED((2, _RROWS, D), jnp.float32),
            pltpu.SemaphoreType.DMA,
            pltpu.SemaphoreType.DMA,
            pltpu.SemaphoreType.DMA,
        ],
    )
    def gather_kernel(idx_hbm, table_hbm, out_hbm, idx_v,
                      gbuf, sbuf0, tbuf0, stage,
                      gsem, psem0, ssem0):
        cid = lax.axis_index("c")
        sid = lax.axis_index("s")
        sbufs = (sbuf0, sbuf0)
        psems = (psem0, psem0)
        tbufs = (tbuf0, tbuf0)
        ssems = (ssem0, ssem0)

        for r in range(_NR):
            pltpu.sync_copy(
                idx_hbm.at[pl.ds(cid * _HALF + r * _RROWS + sid * _C, _C)],
                idx_v.at[r])

        stage_off = pl.multiple_of(sid * _C, 8)
        slab_off = pl.multiple_of(sid * _SLAB, _SLAB)

        def gather(r):
            return pltpu.async_copy(table_hbm.at[idx_v.at[r]], gbuf, gsem)

        def pull(r, q):
            return pltpu.async_copy(
                stage.at[r % 2, pl.ds(q * _Q, _Q), pl.ds(slab_off, _SLAB)],
                sbufs[q % 2], psems[q % 2])

        def store(r, q):
            bcol = pl.multiple_of(cid * _HALF + r * _RROWS + q * _Q, _Q)
            return pltpu.async_copy(
                tbufs[q % 2].at[:, pl.ds(0, _Q)],
                out_hbm.at[pl.ds(slab_off, _SLAB), pl.ds(bcol, _Q)],
                ssems[q % 2])

        rows16 = [lax.iota(jnp.int32, 16) + (j * 16) for j in range(_SLAB // 16)]

        g = gather(0)
        p_pend = [None, None]
        s_pend = [None, None]
        for r in range(_NR):
            g.wait()
            pltpu.sync_copy(gbuf, stage.at[r % 2, pl.ds(stage_off, _C)])
            if r + 1 < _NR:
                g = gather(r + 1)
            plsc.subcore_barrier()

            @pl.when(sid < _NSLAB)
            def _transpose_round(r=r):
                sbuf = sbufs[0]
                tbuf = tbufs[0]
                s_prev = None
                for q in range(_NQ):
                    pull(r, q).wait()
                    if s_prev is not None:
                        s_prev.wait()

                    def body(b, _):
                        col = jnp.full((16,), 0, jnp.int32) + b
                        for j in range(_SLAB // 16):
                            v = sbuf[b, pl.ds(j * 16, 16)]
                            plsc.store_scatter(tbuf, [rows16[j], col], v)
                        return 0

                    lax.fori_loop(0, _Q, body, 0)
                    s_prev = store(r, q)
                s_prev.wait()

    return gather_kernel


_gather = _make_gather()


def kernel(row_idx, emb_weight):
    out_t = _gather(row_idx.astype(jnp.int32), emb_weight)
    return jnp.transpose(out_t.reshape(PROMPT_LEN, DIM, BATCH), (2, 0, 1))


# speedup vs baseline: 3.8004x; 3.8004x over previous
import functools

import jax
import jax.numpy as jnp
from jax import lax
from jax.experimental import pallas as pl
from jax.experimental.pallas import tpu as pltpu
from jax.experimental.pallas import tpu_sc as plsc

DIM = 64
PROMPT_LEN = 20
BATCH = 4096
D = PROMPT_LEN * DIM

_NC = 2
_NS = 16
_NW = _NC * _NS
_BPW = BATCH // _NW
_C = 32
_NCHUNK = _BPW // _C
_NB = 3


def _make_gather():
    mesh = plsc.VectorSubcoreMesh(
        core_axis_name="c", subcore_axis_name="s", num_cores=_NC)

    @functools.partial(
        pl.kernel,
        mesh=mesh,
        out_type=jax.ShapeDtypeStruct((BATCH, D), jnp.float32),
        scratch_types=[
            pltpu.VMEM((_BPW,), jnp.int32),
        ]
        + [pltpu.VMEM((_C, D), jnp.float32) for _ in range(_NB)]
        + [pltpu.SemaphoreType.DMA for _ in range(2 * _NB)],
    )
    def gather_kernel(idx_hbm, table_hbm, out_hbm, idx_v, *rest):
        bufs = rest[:_NB]
        gsems = rest[_NB:2 * _NB]
        osems = rest[2 * _NB:]
        wid = lax.axis_index("s") * _NC + lax.axis_index("c")
        base = wid * _BPW
        pltpu.sync_copy(idx_hbm.at[pl.ds(base, _BPW)], idx_v)

        def gather(c):
            b = c % _NB
            return pltpu.async_copy(
                table_hbm.at[idx_v.at[pl.ds(c * _C, _C)]], bufs[b], gsems[b])

        def store(c):
            b = c % _NB
            return pltpu.async_copy(
                bufs[b], out_hbm.at[pl.ds(base + c * _C, _C)], osems[b])

        g_pend = [None] * _NB
        s_pend = [None] * _NB
        for c in range(min(_NB, _NCHUNK)):
            g_pend[c % _NB] = gather(c)
        for c in range(_NCHUNK):
            b = c % _NB
            g_pend[b].wait()
            g_pend[b] = None
            s_pend[b] = store(c)
            if c + _NB < _NCHUNK:
                s_pend[b].wait()
                s_pend[b] = None
                g_pend[b] = gather(c + _NB)
        for h in s_pend:
            if h is not None:
                h.wait()

    return gather_kernel


_gather = _make_gather()


def kernel(row_idx, emb_weight):
    out = _gather(row_idx.astype(jnp.int32), emb_weight)
    return out.reshape(BATCH, PROMPT_LEN, DIM)


# speedup vs baseline: 3.8188x; 1.0049x over previous
import functools

import jax
import jax.numpy as jnp
from jax import lax
from jax.experimental import pallas as pl
from jax.experimental.pallas import tpu as pltpu
from jax.experimental.pallas import tpu_sc as plsc

DIM = 64
PROMPT_LEN = 20
BATCH = 4096
D = PROMPT_LEN * DIM

_NC = 2
_NS = 16
_NW = _NC * _NS
_BPW = BATCH // _NW
_C = 16
_NCHUNK = _BPW // _C
_NB = 6


def _make_gather():
    mesh = plsc.VectorSubcoreMesh(
        core_axis_name="c", subcore_axis_name="s", num_cores=_NC)

    @functools.partial(
        pl.kernel,
        mesh=mesh,
        out_type=jax.ShapeDtypeStruct((BATCH, D), jnp.float32),
        scratch_types=[
            pltpu.VMEM((_BPW,), jnp.int32),
        ]
        + [pltpu.VMEM((_C, D), jnp.float32) for _ in range(_NB)]
        + [pltpu.SemaphoreType.DMA for _ in range(2 * _NB)],
    )
    def gather_kernel(idx_hbm, table_hbm, out_hbm, idx_v, *rest):
        bufs = rest[:_NB]
        gsems = rest[_NB:2 * _NB]
        osems = rest[2 * _NB:]
        wid = lax.axis_index("s") * _NC + lax.axis_index("c")
        base = wid * _BPW
        pltpu.sync_copy(idx_hbm.at[pl.ds(base, _BPW)], idx_v)

        def gather(c):
            b = c % _NB
            return pltpu.async_copy(
                table_hbm.at[idx_v.at[pl.ds(c * _C, _C)]], bufs[b], gsems[b])

        def store(c):
            b = c % _NB
            return pltpu.async_copy(
                bufs[b], out_hbm.at[pl.ds(base + c * _C, _C)], osems[b])

        g_pend = [None] * _NB
        s_pend = [None] * _NB
        for c in range(min(_NB, _NCHUNK)):
            g_pend[c % _NB] = gather(c)
        for c in range(_NCHUNK):
            b = c % _NB
            g_pend[b].wait()
            g_pend[b] = None
            s_pend[b] = store(c)
            if c + _NB < _NCHUNK:
                s_pend[b].wait()
                s_pend[b] = None
                g_pend[b] = gather(c + _NB)
        for h in s_pend:
            if h is not None:
                h.wait()

    return gather_kernel


_gather = _make_gather()


def kernel(row_idx, emb_weight):
    out = _gather(row_idx.astype(jnp.int32), emb_weight)
    return out.reshape(BATCH, PROMPT_LEN, DIM)
